# Initial kernel scaffold; baseline (speedup 1.0000x reference)
import jax
import jax.numpy as jnp
from jax.experimental import pallas as pl


def kernel(codebooks, codes, scales, L, R):
    raise NotImplementedError("write your pallas kernel here")


# speedup vs baseline: 18.4841x; 18.4841x over previous
import functools

import jax
import jax.numpy as jnp
from jax import lax
from jax.experimental import pallas as pl
from jax.experimental.pallas import tpu as pltpu
from jax.experimental.pallas import tpu_sc as plsc

ROWS, COLS = 4096, 4096
NUM_CODEBOOKS = 2
CODEBOOK_SIZE = 256
CENTROID_LEN = 8
RANK = 256

NC, NS, LANES = 2, 16, 16
NW = NC * NS
N_CODES_TOTAL = ROWS * COLS // CENTROID_LEN
CODES_PER_W = N_CODES_TOTAL // NW
ROWS_PER_W = ROWS // NW
CODES_PER_ROW = COLS // CENTROID_LEN

CODES_PER_CHUNK = 4096
OUT_PER_CHUNK = CODES_PER_CHUNK * CENTROID_LEN
NCHUNK = CODES_PER_W // CODES_PER_CHUNK
GROUPS_PER_CHUNK = CODES_PER_CHUNK // LANES


def _dequant_body(cb_hbm, codes_hbm, scales_hbm, dq_hbm,
                  cb_v, codes_v, out_v, scales_v):
    wid = lax.axis_index("s") * NC + lax.axis_index("c")
    code_base = wid * CODES_PER_W
    row_base = wid * ROWS_PER_W
    out_base = code_base * CENTROID_LEN
    half_off = (wid // (NW // NUM_CODEBOOKS)) * (CODEBOOK_SIZE * CENTROID_LEN)

    pltpu.sync_copy(cb_hbm, cb_v)
    pltpu.sync_copy(scales_hbm.at[pl.ds(row_base, ROWS_PER_W)], scales_v)

    off8 = lax.iota(jnp.int32, LANES) * CENTROID_LEN

    def chunk(ci, carry):
        pltpu.sync_copy(
            codes_hbm.at[pl.ds(code_base + ci * CODES_PER_CHUNK,
                               CODES_PER_CHUNK)],
            codes_v)

        def group(gi, c2):
            codes16 = codes_v[pl.ds(gi * LANES, LANES)]
            row_in_w = ci * (CODES_PER_CHUNK // CODES_PER_ROW) + \
                (gi // (CODES_PER_ROW // LANES))
            scale = plsc.load_gather(
                scales_v, [jnp.full((LANES,), row_in_w, jnp.int32)])
            idx8 = codes16 * CENTROID_LEN + half_off
            obase = gi * (LANES * CENTROID_LEN)
            for k in range(CENTROID_LEN):
                vals = plsc.load_gather(cb_v, [idx8 + k])
                plsc.store_scatter(out_v, [off8 + (obase + k)], vals * scale)
            return c2

        lax.fori_loop(0, GROUPS_PER_CHUNK, group, 0, unroll=False)
        pltpu.sync_copy(
            out_v,
            dq_hbm.at[pl.ds(out_base + ci * OUT_PER_CHUNK, OUT_PER_CHUNK)])
        return carry

    lax.fori_loop(0, NCHUNK, chunk, 0, unroll=False)


_dequant_sc = functools.partial(
    pl.kernel,
    out_type=jax.ShapeDtypeStruct((N_CODES_TOTAL * CENTROID_LEN,),
                                  jnp.float32),
    mesh=plsc.VectorSubcoreMesh(core_axis_name="c", subcore_axis_name="s"),
    compiler_params=pltpu.CompilerParams(needs_layout_passes=False),
    scratch_types=[
        pltpu.VMEM((NUM_CODEBOOKS * CODEBOOK_SIZE * CENTROID_LEN,),
                   jnp.float32),
        pltpu.VMEM((CODES_PER_CHUNK,), jnp.int32),
        pltpu.VMEM((OUT_PER_CHUNK,), jnp.float32),
        pltpu.VMEM((ROWS_PER_W,), jnp.float32),
    ],
)(_dequant_body)


BM, BN = 1024, 1024


def _addmm_body(dq_ref, l_ref, r_ref, o_ref):
    lr = jnp.dot(l_ref[...].astype(jnp.bfloat16),
                 r_ref[...].astype(jnp.bfloat16),
                 preferred_element_type=jnp.float32)
    o_ref[...] = dq_ref[...] + lr


def _addmm_tc(dq, L, R):
    return pl.pallas_call(
        _addmm_body,
        grid=(ROWS // BM, COLS // BN),
        in_specs=[
            pl.BlockSpec((BM, BN), lambda i, j: (i, j)),
            pl.BlockSpec((BM, RANK), lambda i, j: (i, 0)),
            pl.BlockSpec((RANK, BN), lambda i, j: (0, j)),
        ],
        out_specs=pl.BlockSpec((BM, BN), lambda i, j: (i, j)),
        out_shape=jax.ShapeDtypeStruct((ROWS, COLS), jnp.float32),
        compiler_params=pltpu.CompilerParams(
            dimension_semantics=("parallel", "parallel")),
    )(dq, L, R)


def kernel(codebooks, codes, scales, L, R):
    cb_flat = codebooks.reshape(-1)
    codes_flat = codes.reshape(-1)
    scales_flat = scales.reshape(-1)
    dq = _dequant_sc(cb_flat, codes_flat, scales_flat)
    return _addmm_tc(dq.reshape(ROWS, COLS), L, R)


# speedup vs baseline: 20.8161x; 1.1262x over previous
import functools

import jax
import jax.numpy as jnp
from jax import lax
from jax.experimental import pallas as pl
from jax.experimental.pallas import tpu as pltpu
from jax.experimental.pallas import tpu_sc as plsc

ROWS, COLS = 4096, 4096
NUM_CODEBOOKS = 2
CODEBOOK_SIZE = 256
CENTROID_LEN = 8
RANK = 256

NC, NS, LANES = 2, 16, 16
NW = NC * NS
N_CODES_TOTAL = ROWS * COLS // CENTROID_LEN
CODES_PER_W = N_CODES_TOTAL // NW

CODES_PER_CHUNK = 4096
OUT_PER_CHUNK = CODES_PER_CHUNK * CENTROID_LEN
NCHUNK = CODES_PER_W // CODES_PER_CHUNK
GROUPS_PER_CHUNK = CODES_PER_CHUNK // LANES
TABLE_LEN = NUM_CODEBOOKS * CODEBOOK_SIZE * CENTROID_LEN


def _dequant_body(cb_hbm, codes_hbm, dq_hbm,
                  cb_v, codes_v0, codes_v1, out_v0, out_v1,
                  sem_cb, sem_in0, sem_in1, sem_out0, sem_out1):
    wid = lax.axis_index("s") * NC + lax.axis_index("c")
    code_base = wid * CODES_PER_W
    out_base = code_base * CENTROID_LEN
    half_off = (wid // (NW // NUM_CODEBOOKS)) * (CODEBOOK_SIZE * CENTROID_LEN)

    pltpu.async_copy(cb_hbm, cb_v, sem_cb).wait()
    off8 = lax.iota(jnp.int32, LANES) * CENTROID_LEN

    sem_in = [sem_in0, sem_in1]
    sem_out = [sem_out0, sem_out1]
    codes_v = [codes_v0, codes_v1]
    out_v = [out_v0, out_v1]

    def start_in(ci):
        return pltpu.async_copy(
            codes_hbm.at[pl.ds(code_base + ci * CODES_PER_CHUNK,
                               CODES_PER_CHUNK)],
            codes_v[ci % 2], sem_in[ci % 2])

    def start_out(ci):
        return pltpu.async_copy(
            out_v[ci % 2],
            dq_hbm.at[pl.ds(out_base + ci * OUT_PER_CHUNK, OUT_PER_CHUNK)],
            sem_out[ci % 2])

    in_d = {0: start_in(0), 1: start_in(1)}
    out_d = {}
    for ci in range(NCHUNK):
        b = ci % 2
        in_d[ci].wait()
        if ci - 2 in out_d:
            out_d[ci - 2].wait()
        codes_b = codes_v[b]
        out_b = out_v[b]

        def group(gi, c2):
            codes16 = codes_b[pl.ds(gi * LANES, LANES)]
            eidx = codes16 * CENTROID_LEN + half_off
            obase = gi * (LANES * CENTROID_LEN)
            for k in range(CENTROID_LEN):
                vals = plsc.load_gather(cb_v, [eidx + k])
                plsc.store_scatter(out_b, [off8 + (obase + k)], vals)
            return c2

        lax.fori_loop(0, GROUPS_PER_CHUNK, group, 0, unroll=False)

        out_d[ci] = start_out(ci)
        if ci + 2 < NCHUNK:
            in_d[ci + 2] = start_in(ci + 2)
    out_d[NCHUNK - 2].wait()
    out_d[NCHUNK - 1].wait()


_dequant_sc = functools.partial(
    pl.kernel,
    out_type=jax.ShapeDtypeStruct((ROWS * COLS,), jnp.float32),
    mesh=plsc.VectorSubcoreMesh(core_axis_name="c", subcore_axis_name="s"),
    compiler_params=pltpu.CompilerParams(needs_layout_passes=False),
    scratch_types=[
        pltpu.VMEM((TABLE_LEN,), jnp.float32),
        pltpu.VMEM((CODES_PER_CHUNK,), jnp.int32),
        pltpu.VMEM((CODES_PER_CHUNK,), jnp.int32),
        pltpu.VMEM((OUT_PER_CHUNK,), jnp.float32),
        pltpu.VMEM((OUT_PER_CHUNK,), jnp.float32),
        pltpu.SemaphoreType.DMA,
        pltpu.SemaphoreType.DMA,
        pltpu.SemaphoreType.DMA,
        pltpu.SemaphoreType.DMA,
        pltpu.SemaphoreType.DMA,
    ],
)(_dequant_body)


BM, BN = 512, 1024


def _addmm_body(dq_ref, sc_ref, l_ref, r_ref, o_ref):
    lr = jnp.dot(l_ref[...].astype(jnp.bfloat16),
                 r_ref[...].astype(jnp.bfloat16),
                 preferred_element_type=jnp.float32)
    o_ref[...] = dq_ref[...] * sc_ref[...] + lr


def _addmm_tc(dq, scales, L, R):
    return pl.pallas_call(
        _addmm_body,
        grid=(ROWS // BM, COLS // BN),
        in_specs=[
            pl.BlockSpec((BM, BN), lambda i, j: (i, j)),
            pl.BlockSpec((BM, 1), lambda i, j: (i, 0)),
            pl.BlockSpec((BM, RANK), lambda i, j: (i, 0)),
            pl.BlockSpec((RANK, BN), lambda i, j: (0, j)),
        ],
        out_specs=pl.BlockSpec((BM, BN), lambda i, j: (i, j)),
        out_shape=jax.ShapeDtypeStruct((ROWS, COLS), jnp.float32),
        compiler_params=pltpu.CompilerParams(
            dimension_semantics=("parallel", "parallel")),
    )(dq, scales, L, R)


def kernel(codebooks, codes, scales, L, R):
    cb_flat = codebooks.reshape(-1)
    codes_flat = codes.reshape(-1)
    dq = _dequant_sc(cb_flat, codes_flat)
    return _addmm_tc(dq.reshape(ROWS, COLS), scales, L, R)


# speedup vs baseline: 43.2462x; 2.0775x over previous
import functools

import jax
import jax.numpy as jnp
from jax import lax
from jax.experimental import pallas as pl
from jax.experimental.pallas import tpu as pltpu
from jax.experimental.pallas import tpu_sc as plsc

ROWS, COLS = 4096, 4096
NUM_CODEBOOKS = 2
CODEBOOK_SIZE = 256
CENTROID_LEN = 8
RANK = 256

NC, NS, LANES = 2, 16, 16
NW = NC * NS
HROWS = ROWS // NUM_CODEBOOKS
N_CODES_HALF = HROWS * COLS // CENTROID_LEN
CODES_PER_W = N_CODES_HALF // NW

CODES_PER_CHUNK = 4096
OUT_PER_CHUNK = CODES_PER_CHUNK * CENTROID_LEN
NCHUNK = CODES_PER_W // CODES_PER_CHUNK
GROUPS_PER_CHUNK = CODES_PER_CHUNK // LANES
TABLE_LEN = NUM_CODEBOOKS * CODEBOOK_SIZE * CENTROID_LEN


def _make_dequant_half(half):
    half_off = half * CODEBOOK_SIZE * CENTROID_LEN

    def body(cb_hbm, codes_hbm, dq_hbm,
             cb_v, codes_v0, codes_v1, out_v0, out_v1,
             sem_cb, sem_in0, sem_in1, sem_out0, sem_out1):
        wid = lax.axis_index("s") * NC + lax.axis_index("c")
        code_base = wid * CODES_PER_W
        out_base = code_base * CENTROID_LEN

        pltpu.async_copy(cb_hbm, cb_v, sem_cb).wait()
        off8 = lax.iota(jnp.int32, LANES) * CENTROID_LEN

        sem_in = [sem_in0, sem_in1]
        sem_out = [sem_out0, sem_out1]
        codes_v = [codes_v0, codes_v1]
        out_v = [out_v0, out_v1]

        def start_in(ci):
            return pltpu.async_copy(
                codes_hbm.at[pl.ds(code_base + ci * CODES_PER_CHUNK,
                                   CODES_PER_CHUNK)],
                codes_v[ci % 2], sem_in[ci % 2])

        def start_out(ci):
            return pltpu.async_copy(
                out_v[ci % 2],
                dq_hbm.at[pl.ds(out_base + ci * OUT_PER_CHUNK,
                                OUT_PER_CHUNK)],
                sem_out[ci % 2])

        in_d = {0: start_in(0), 1: start_in(1)}
        out_d = {}
        for ci in range(NCHUNK):
            b = ci % 2
            in_d[ci].wait()
            if ci - 2 in out_d:
                out_d[ci - 2].wait()
            codes_b = codes_v[b]
            out_b = out_v[b]

            @plsc.parallel_loop(0, GROUPS_PER_CHUNK, unroll=4)
            def group(gi):
                codes16 = codes_b[pl.ds(gi * LANES, LANES)]
                eidx = codes16 * CENTROID_LEN + half_off
                obase = gi * (LANES * CENTROID_LEN)
                for k in range(CENTROID_LEN):
                    vals = plsc.load_gather(cb_v, [eidx + k])
                    plsc.store_scatter(out_b, [off8 + (obase + k)], vals)

            out_d[ci] = start_out(ci)
            if ci + 2 < NCHUNK:
                in_d[ci + 2] = start_in(ci + 2)
        out_d[NCHUNK - 2].wait()
        out_d[NCHUNK - 1].wait()

    return functools.partial(
        pl.kernel,
        out_type=jax.ShapeDtypeStruct((HROWS * COLS,), jnp.float32),
        mesh=plsc.VectorSubcoreMesh(core_axis_name="c",
                                    subcore_axis_name="s"),
        compiler_params=pltpu.CompilerParams(needs_layout_passes=False),
        scratch_types=[
            pltpu.VMEM((TABLE_LEN,), jnp.float32),
            pltpu.VMEM((CODES_PER_CHUNK,), jnp.int32),
            pltpu.VMEM((CODES_PER_CHUNK,), jnp.int32),
            pltpu.VMEM((OUT_PER_CHUNK,), jnp.float32),
            pltpu.VMEM((OUT_PER_CHUNK,), jnp.float32),
            pltpu.SemaphoreType.DMA,
            pltpu.SemaphoreType.DMA,
            pltpu.SemaphoreType.DMA,
            pltpu.SemaphoreType.DMA,
            pltpu.SemaphoreType.DMA,
        ],
    )(body)


_dequant_sc = [_make_dequant_half(0), _make_dequant_half(1)]


BM, BN = 512, 2048


def _addmm_body0(dq_ref, sc_ref, l_ref, r_ref, o_ref):
    lr = jnp.dot(l_ref[...].astype(jnp.bfloat16),
                 r_ref[...].astype(jnp.bfloat16),
                 preferred_element_type=jnp.float32)
    o_ref[...] = dq_ref[...] * sc_ref[...] + lr


def _addmm_body1(prev_ref, dq_ref, sc_ref, l_ref, r_ref, o_ref):
    del prev_ref
    _addmm_body0(dq_ref, sc_ref, l_ref, r_ref, o_ref)


def _addmm_tc(half, dq, scales_h, L_h, R, prev=None):
    data_specs = [
        pl.BlockSpec((BM, BN), lambda i, j: (i, j)),
        pl.BlockSpec((BM, 1), lambda i, j: (i, 0)),
        pl.BlockSpec((BM, RANK), lambda i, j: (i, 0)),
        pl.BlockSpec((RANK, BN), lambda i, j: (0, j)),
    ]
    row_off = half * (HROWS // BM)
    kwargs = {}
    if half == 0:
        body = _addmm_body0
        in_specs = data_specs
        args = (dq, scales_h, L_h, R)
    else:
        body = _addmm_body1
        in_specs = [pl.BlockSpec(memory_space=pltpu.MemorySpace.HBM)] + data_specs
        args = (prev, dq, scales_h, L_h, R)
        kwargs["input_output_aliases"] = {0: 0}
    return pl.pallas_call(
        body,
        grid=(HROWS // BM, COLS // BN),
        in_specs=in_specs,
        out_specs=pl.BlockSpec((BM, BN), lambda i, j: (i + row_off, j)),
        out_shape=jax.ShapeDtypeStruct((ROWS, COLS), jnp.float32),
        compiler_params=pltpu.CompilerParams(
            dimension_semantics=("parallel", "parallel")),
        **kwargs,
    )(*args)


def kernel(codebooks, codes, scales, L, R):
    cb_flat = codebooks.reshape(-1)
    dq0 = _dequant_sc[0](cb_flat, codes[0])
    dq1 = _dequant_sc[1](cb_flat, codes[1])
    out = _addmm_tc(0, dq0.reshape(HROWS, COLS), scales[:HROWS],
                    L[:HROWS], R)
    out = _addmm_tc(1, dq1.reshape(HROWS, COLS), scales[HROWS:],
                    L[HROWS:], R, prev=out)
    return out


# speedup vs baseline: 57.5633x; 1.3311x over previous
import functools

import jax
import jax.numpy as jnp
from jax import lax
from jax.experimental import pallas as pl
from jax.experimental.pallas import tpu as pltpu
from jax.experimental.pallas import tpu_sc as plsc

ROWS, COLS = 4096, 4096
NUM_CODEBOOKS = 2
CODEBOOK_SIZE = 256
CENTROID_LEN = 8
RANK = 256

NC, NS, LANES = 2, 16, 16
NW = NC * NS
HROWS = ROWS // NUM_CODEBOOKS
N_CODES_HALF = HROWS * COLS // CENTROID_LEN
CODES_PER_W = N_CODES_HALF // NW

CODES_PER_CHUNK = 4096
CHUNK_ROWS = CODES_PER_CHUNK * CENTROID_LEN // COLS
NCHUNK = CODES_PER_W // CODES_PER_CHUNK
GROUPS_PER_CHUNK = CODES_PER_CHUNK // LANES
GROUPS_PER_ROW = COLS // (LANES * CENTROID_LEN)
ROWS_PER_W = HROWS // NW
TABLE_LEN = NUM_CODEBOOKS * CODEBOOK_SIZE * CENTROID_LEN


def _make_dequant_half(half):
    half_off = half * CODEBOOK_SIZE * CENTROID_LEN

    def body(cb_hbm, codes_hbm, dq_hbm,
             cb_v, codes_v0, codes_v1, out_v0, out_v1,
             sem_cb, sem_in0, sem_in1, sem_out0, sem_out1):
        wid = lax.axis_index("s") * NC + lax.axis_index("c")
        code_base = wid * CODES_PER_W
        row_base = wid * ROWS_PER_W

        pltpu.async_copy(cb_hbm, cb_v, sem_cb).wait()
        off8 = lax.iota(jnp.int32, LANES) * CENTROID_LEN

        sem_in = [sem_in0, sem_in1]
        sem_out = [sem_out0, sem_out1]
        codes_v = [codes_v0, codes_v1]
        out_v = [out_v0, out_v1]

        def start_in(ci):
            return pltpu.async_copy(
                codes_hbm.at[pl.ds(code_base + ci * CODES_PER_CHUNK,
                                   CODES_PER_CHUNK)],
                codes_v[ci % 2], sem_in[ci % 2])

        def start_out(ci):
            return pltpu.async_copy(
                out_v[ci % 2],
                dq_hbm.at[pl.ds(row_base + ci * CHUNK_ROWS, CHUNK_ROWS)],
                sem_out[ci % 2])

        in_d = {0: start_in(0), 1: start_in(1)}
        out_d = {}
        for ci in range(NCHUNK):
            b = ci % 2
            in_d[ci].wait()
            if ci - 2 in out_d:
                out_d[ci - 2].wait()
            codes_b = codes_v[b]
            out_b = out_v[b]

            @plsc.parallel_loop(0, GROUPS_PER_CHUNK, unroll=4)
            def group(gi):
                codes16 = codes_b[pl.ds(gi * LANES, LANES)]
                eidx = codes16 * CENTROID_LEN + half_off
                row16 = jnp.full((LANES,), gi >> 5, jnp.int32)
                cbase = (gi & (GROUPS_PER_ROW - 1)) * (LANES * CENTROID_LEN)
                for k in range(CENTROID_LEN):
                    vals = plsc.load_gather(cb_v, [eidx + k])
                    plsc.store_scatter(
                        out_b, [row16, off8 + (cbase + k)], vals)

            out_d[ci] = start_out(ci)
            if ci + 2 < NCHUNK:
                in_d[ci + 2] = start_in(ci + 2)
        out_d[NCHUNK - 2].wait()
        out_d[NCHUNK - 1].wait()

    return functools.partial(
        pl.kernel,
        out_type=jax.ShapeDtypeStruct((HROWS, COLS), jnp.float32),
        mesh=plsc.VectorSubcoreMesh(core_axis_name="c",
                                    subcore_axis_name="s"),
        compiler_params=pltpu.CompilerParams(needs_layout_passes=False),
        scratch_types=[
            pltpu.VMEM((TABLE_LEN,), jnp.float32),
            pltpu.VMEM((CODES_PER_CHUNK,), jnp.int32),
            pltpu.VMEM((CODES_PER_CHUNK,), jnp.int32),
            pltpu.VMEM((CHUNK_ROWS, COLS), jnp.float32),
            pltpu.VMEM((CHUNK_ROWS, COLS), jnp.float32),
            pltpu.SemaphoreType.DMA,
            pltpu.SemaphoreType.DMA,
            pltpu.SemaphoreType.DMA,
            pltpu.SemaphoreType.DMA,
            pltpu.SemaphoreType.DMA,
        ],
    )(body)


_dequant_sc = [_make_dequant_half(0), _make_dequant_half(1)]


BM, BN = 512, 2048


def _addmm_body0(dq_ref, sc_ref, l_ref, r_ref, o_ref):
    lr = jnp.dot(l_ref[...].astype(jnp.bfloat16),
                 r_ref[...].astype(jnp.bfloat16),
                 preferred_element_type=jnp.float32)
    o_ref[...] = dq_ref[...] * sc_ref[...] + lr


def _addmm_body1(prev_ref, dq_ref, sc_ref, l_ref, r_ref, o_ref):
    del prev_ref
    _addmm_body0(dq_ref, sc_ref, l_ref, r_ref, o_ref)


def _addmm_tc(half, dq, scales_h, L_h, R, prev=None):
    data_specs = [
        pl.BlockSpec((BM, BN), lambda i, j: (i, j)),
        pl.BlockSpec((BM, 1), lambda i, j: (i, 0)),
        pl.BlockSpec((BM, RANK), lambda i, j: (i, 0)),
        pl.BlockSpec((RANK, BN), lambda i, j: (0, j)),
    ]
    row_off = half * (HROWS // BM)
    kwargs = {}
    if half == 0:
        body = _addmm_body0
        in_specs = data_specs
        args = (dq, scales_h, L_h, R)
    else:
        body = _addmm_body1
        in_specs = [pl.BlockSpec(memory_space=pltpu.MemorySpace.HBM)] + data_specs
        args = (prev, dq, scales_h, L_h, R)
        kwargs["input_output_aliases"] = {0: 0}
    return pl.pallas_call(
        body,
        grid=(HROWS // BM, COLS // BN),
        in_specs=in_specs,
        out_specs=pl.BlockSpec((BM, BN), lambda i, j: (i + row_off, j)),
        out_shape=jax.ShapeDtypeStruct((ROWS, COLS), jnp.float32),
        compiler_params=pltpu.CompilerParams(
            dimension_semantics=("parallel", "parallel")),
        **kwargs,
    )(*args)


def kernel(codebooks, codes, scales, L, R):
    cb_flat = codebooks.reshape(-1)
    dq0 = _dequant_sc[0](cb_flat, codes[0])
    dq1 = _dequant_sc[1](cb_flat, codes[1])
    out = _addmm_tc(0, dq0, scales[:HROWS], L[:HROWS], R)
    out = _addmm_tc(1, dq1, scales[HROWS:], L[HROWS:], R, prev=out)
    return out


# speedup vs baseline: 63.6260x; 1.1053x over previous
import functools

import jax
import jax.numpy as jnp
from jax import lax
from jax.experimental import pallas as pl
from jax.experimental.pallas import tpu as pltpu
from jax.experimental.pallas import tpu_sc as plsc

ROWS, COLS = 4096, 4096
NUM_CODEBOOKS = 2
CODEBOOK_SIZE = 256
CENTROID_LEN = 8
RANK = 256

NC, NS, LANES = 2, 16, 16
NW = NC * NS
HROWS = ROWS // NUM_CODEBOOKS
N_CODES_HALF = HROWS * COLS // CENTROID_LEN
CODES_PER_W = N_CODES_HALF // NW

CODES_PER_CHUNK = 4096
CHUNK_ROWS = CODES_PER_CHUNK * CENTROID_LEN // COLS
NCHUNK = CODES_PER_W // CODES_PER_CHUNK
GROUPS_PER_CHUNK = CODES_PER_CHUNK // LANES
GROUPS_PER_ROW = COLS // (LANES * CENTROID_LEN)
ROWS_PER_W = HROWS // NW
TABLE_LEN = NUM_CODEBOOKS * CODEBOOK_SIZE * CENTROID_LEN


def _make_dequant_half(half):
    half_off = half * CODEBOOK_SIZE * CENTROID_LEN

    def body(cb_hbm, codes_hbm, dq_hbm,
             cb_v, codes_v0, codes_v1, out_v0, out_v1,
             sem_cb, sem_in0, sem_in1, sem_out0, sem_out1):
        wid = lax.axis_index("s") * NC + lax.axis_index("c")
        code_base = wid * CODES_PER_W
        row_base = wid * ROWS_PER_W

        pltpu.async_copy(cb_hbm, cb_v, sem_cb).wait()
        off8 = lax.iota(jnp.int32, LANES) * CENTROID_LEN

        sem_in = [sem_in0, sem_in1]
        sem_out = [sem_out0, sem_out1]
        codes_v = [codes_v0, codes_v1]
        out_v = [out_v0, out_v1]

        def start_in(ci):
            return pltpu.async_copy(
                codes_hbm.at[half,
                             pl.ds(code_base + ci * CODES_PER_CHUNK,
                                   CODES_PER_CHUNK)],
                codes_v[ci % 2], sem_in[ci % 2])

        def start_out(ci):
            return pltpu.async_copy(
                out_v[ci % 2],
                dq_hbm.at[pl.ds(row_base + ci * CHUNK_ROWS, CHUNK_ROWS)],
                sem_out[ci % 2])

        in_d = {0: start_in(0), 1: start_in(1)}
        out_d = {}
        for ci in range(NCHUNK):
            b = ci % 2
            in_d[ci].wait()
            if ci - 2 in out_d:
                out_d[ci - 2].wait()
            codes_b = codes_v[b]
            out_b = out_v[b]

            @plsc.parallel_loop(0, GROUPS_PER_CHUNK, unroll=4)
            def group(gi):
                codes16 = codes_b[pl.ds(gi * LANES, LANES)]
                eidx = codes16 * CENTROID_LEN + half_off
                row16 = jnp.full((LANES,), gi >> 5, jnp.int32)
                cbase = (gi & (GROUPS_PER_ROW - 1)) * (LANES * CENTROID_LEN)
                for k in range(CENTROID_LEN):
                    vals = plsc.load_gather(cb_v, [eidx + k])
                    plsc.store_scatter(
                        out_b, [row16, off8 + (cbase + k)], vals)

            out_d[ci] = start_out(ci)
            if ci + 2 < NCHUNK:
                in_d[ci + 2] = start_in(ci + 2)
        out_d[NCHUNK - 2].wait()
        out_d[NCHUNK - 1].wait()

    return functools.partial(
        pl.kernel,
        out_type=jax.ShapeDtypeStruct((HROWS, COLS), jnp.float32),
        mesh=plsc.VectorSubcoreMesh(core_axis_name="c",
                                    subcore_axis_name="s"),
        compiler_params=pltpu.CompilerParams(needs_layout_passes=False),
        scratch_types=[
            pltpu.VMEM((TABLE_LEN,), jnp.float32),
            pltpu.VMEM((CODES_PER_CHUNK,), jnp.int32),
            pltpu.VMEM((CODES_PER_CHUNK,), jnp.int32),
            pltpu.VMEM((CHUNK_ROWS, COLS), jnp.float32),
            pltpu.VMEM((CHUNK_ROWS, COLS), jnp.float32),
            pltpu.SemaphoreType.DMA,
            pltpu.SemaphoreType.DMA,
            pltpu.SemaphoreType.DMA,
            pltpu.SemaphoreType.DMA,
            pltpu.SemaphoreType.DMA,
        ],
    )(body)


_dequant_sc = [_make_dequant_half(0), _make_dequant_half(1)]


BM, BN = 512, 2048


def _addmm_body0(dq_ref, sc_ref, l_ref, r_ref, o_ref):
    lr = jnp.dot(l_ref[...], r_ref[...],
                 preferred_element_type=jnp.float32)
    o_ref[...] = dq_ref[...] * sc_ref[...] + lr


def _addmm_body1(prev_ref, dq_ref, sc_ref, l_ref, r_ref, o_ref):
    del prev_ref
    _addmm_body0(dq_ref, sc_ref, l_ref, r_ref, o_ref)


def _addmm_tc(half, dq, scales, L16, R16, prev=None):
    row_off = half * (HROWS // BM)
    data_specs = [
        pl.BlockSpec((BM, BN), lambda i, j: (i, j)),
        pl.BlockSpec((BM, 1), lambda i, j: (i + row_off, 0)),
        pl.BlockSpec((BM, RANK), lambda i, j: (i + row_off, 0)),
        pl.BlockSpec((RANK, BN), lambda i, j: (0, j)),
    ]
    kwargs = {}
    if half == 0:
        body = _addmm_body0
        in_specs = data_specs
        args = (dq, scales, L16, R16)
    else:
        body = _addmm_body1
        in_specs = [pl.BlockSpec(memory_space=pltpu.MemorySpace.HBM)] + data_specs
        args = (prev, dq, scales, L16, R16)
        kwargs["input_output_aliases"] = {0: 0}
    return pl.pallas_call(
        body,
        grid=(HROWS // BM, COLS // BN),
        in_specs=in_specs,
        out_specs=pl.BlockSpec((BM, BN), lambda i, j: (i + row_off, j)),
        out_shape=jax.ShapeDtypeStruct((ROWS, COLS), jnp.float32),
        compiler_params=pltpu.CompilerParams(
            dimension_semantics=("parallel", "parallel")),
        **kwargs,
    )(*args)


def kernel(codebooks, codes, scales, L, R):
    cb_flat = codebooks.reshape(-1)
    L16 = L.astype(jnp.bfloat16)
    R16 = R.astype(jnp.bfloat16)
    dq0 = _dequant_sc[0](cb_flat, codes)
    dq1 = _dequant_sc[1](cb_flat, codes)
    out = _addmm_tc(0, dq0, scales, L16, R16)
    out = _addmm_tc(1, dq1, scales, L16, R16, prev=out)
    return out
